# baseline (speedup 1.0000x reference)
import functools

import jax
import jax.numpy as jnp
from jax import lax
from jax.experimental import pallas as pl
from jax.experimental.pallas import tpu as pltpu
from jax.experimental.pallas import tpu_sc as plsc

_N = 100000
_D = 64
_NC, _NS = 2, 16
_NW = _NC * _NS
_CHUNK = 128
_NCHUNKS = (_N + _CHUNK - 1) // _CHUNK
_N_PAD = _NCHUNKS * _CHUNK
_CPW = 25
_LASTBASE = _NCHUNKS - _CPW


def _repack_body(wt_ref, p_ref):
    t = jnp.transpose(wt_ref[...])
    p_ref[...] = jnp.concatenate(
        [t, jnp.zeros_like(t)], axis=1)


_PB = 1024


@jax.jit
def _repack(wt):
    return pl.pallas_call(
        _repack_body,
        grid=((_N + _PB - 1) // _PB,),
        in_specs=[pl.BlockSpec((_D, _PB), lambda j: (0, j))],
        out_specs=pl.BlockSpec((_PB, 128), lambda j: (j, 0)),
        out_shape=jax.ShapeDtypeStruct((_N, 128), jnp.float32),
    )(wt)


@functools.partial(
    pl.kernel,
    out_type=jax.ShapeDtypeStruct((_D, _N), jnp.float32),
    mesh=plsc.VectorSubcoreMesh(core_axis_name="c", subcore_axis_name="s"),
    scratch_types=[
        pltpu.VMEM((_CPW * _CHUNK,), jnp.int32),
        pltpu.VMEM((2, _CHUNK, 128), jnp.float32),
        pltpu.VMEM((2, _D, _CHUNK), jnp.float32),
        pltpu.SemaphoreType.DMA((2,)),
        pltpu.SemaphoreType.DMA((2,)),
    ],
    compiler_params=pltpu.CompilerParams(
        use_tc_tiling_on_sc=True, needs_layout_passes=False),
)
def _emb_lookup(idx_hbm, p_hbm, out_hbm, idx_v, rows_v, oblk_v,
                gsem, ssem):
    wid = lax.axis_index("s") * _NC + lax.axis_index("c")
    base = jnp.minimum((wid * _NCHUNKS) // _NW, _LASTBASE)
    pltpu.sync_copy(idx_hbm.at[pl.ds(base * _CHUNK, _CPW * _CHUNK)], idx_v)

    def prep_and_gather(c, b):
        pltpu.async_copy(
            p_hbm.at[idx_v.at[pl.ds(c * _CHUNK, _CHUNK)]],
            rows_v.at[b], gsem.at[b])

    def gather_wait(b):
        pltpu.make_async_copy(
            p_hbm.at[idx_v.at[pl.ds(0, _CHUNK)]],
            rows_v.at[b], gsem.at[b]).wait()

    def transpose(c, b):
        rows = rows_v.at[b]
        for v in range(_CHUNK // 16):
            lvec = jax.lax.iota(jnp.int32, 16) + v * 16
            for d in range(_D):
                vec = plsc.load_gather(rows, [lvec, jnp.full((16,), d,
                                                             jnp.int32)])
                oblk_v[b, d, pl.ds(v * 16, 16)] = vec

    def store(c, b):
        pltpu.async_copy(
            oblk_v.at[b], out_hbm.at[:, pl.ds((base + c) * _CHUNK, _CHUNK)],
            ssem.at[b])

    def store_wait(b):
        pltpu.make_async_copy(
            out_hbm.at[:, pl.ds(0, _CHUNK)], oblk_v.at[b], ssem.at[b]).wait()

    prep_and_gather(0, 0)

    def body(c, carry):
        b = lax.rem(c, 2)
        nb = 1 - b

        @pl.when(c + 1 < _CPW)
        def _():
            prep_and_gather(c + 1, nb)

        gather_wait(b)

        @pl.when(c >= 2)
        def _():
            store_wait(b)

        transpose(c, b)
        store(c, b)
        return carry

    lax.fori_loop(0, _CPW, body, 0)
    store_wait(0)
    store_wait(1)


def _impl(node_feature, weight):
    idx = jnp.pad(node_feature[:, 0], (0, _N_PAD - _N))
    p = _repack(weight.T)
    return _emb_lookup(idx, p).T


def kernel(node_feature, weight):
    return jax.jit(_impl)(node_feature, weight)


# speedup vs baseline: 1.1139x; 1.1139x over previous
import functools

import jax
import jax.numpy as jnp
from jax import lax
from jax.experimental import pallas as pl
from jax.experimental.pallas import tpu as pltpu
from jax.experimental.pallas import tpu_sc as plsc

_N = 100000
_D = 64
_NC, _NS = 2, 16
_NW = _NC * _NS
_CHUNK = 128
_NCHUNKS = (_N + _CHUNK - 1) // _CHUNK
_N_PAD = _NCHUNKS * _CHUNK
_CPW = 25
_LASTBASE = _NCHUNKS - _CPW


def _repack_body(wt_ref, p_ref):
    x = wt_ref[...]
    eye = jnp.eye(_D, dtype=jnp.float32)
    t = jax.lax.dot_general(x, eye, (((0,), (0,)), ((), ())),
                            preferred_element_type=jnp.float32)
    p_ref[...] = jnp.concatenate([t, jnp.zeros_like(t)], axis=1)


_PB = 1024


@jax.jit
def _repack(wt):
    return pl.pallas_call(
        _repack_body,
        grid=((_N + _PB - 1) // _PB,),
        in_specs=[pl.BlockSpec((_D, _PB), lambda j: (0, j))],
        out_specs=pl.BlockSpec((_PB, 128), lambda j: (j, 0)),
        out_shape=jax.ShapeDtypeStruct((_N, 128), jnp.float32),
    )(wt)


@functools.partial(
    pl.kernel,
    out_type=jax.ShapeDtypeStruct((_D, _N), jnp.float32),
    mesh=plsc.VectorSubcoreMesh(core_axis_name="c", subcore_axis_name="s"),
    scratch_types=[
        pltpu.VMEM((_CPW * _CHUNK,), jnp.int32),
        pltpu.VMEM((2, _CHUNK, 128), jnp.float32),
        pltpu.VMEM((2, _D, _CHUNK), jnp.float32),
        pltpu.SemaphoreType.DMA((2,)),
        pltpu.SemaphoreType.DMA((2,)),
    ],
    compiler_params=pltpu.CompilerParams(
        use_tc_tiling_on_sc=True, needs_layout_passes=False),
)
def _emb_lookup(idx_hbm, p_hbm, out_hbm, idx_v, rows_v, oblk_v,
                gsem, ssem):
    wid = lax.axis_index("s") * _NC + lax.axis_index("c")
    base = jnp.minimum((wid * _NCHUNKS) // _NW, _LASTBASE)
    pltpu.sync_copy(idx_hbm.at[pl.ds(base * _CHUNK, _CPW * _CHUNK)], idx_v)

    def prep_and_gather(c, b):
        pltpu.async_copy(
            p_hbm.at[idx_v.at[pl.ds(c * _CHUNK, _CHUNK)]],
            rows_v.at[b], gsem.at[b])

    def gather_wait(b):
        pltpu.make_async_copy(
            p_hbm.at[idx_v.at[pl.ds(0, _CHUNK)]],
            rows_v.at[b], gsem.at[b]).wait()

    def transpose(c, b):
        oblk = oblk_v.at[b]
        iota = jax.lax.iota(jnp.int32, 16)
        for k in range(_D // 16):
            dvec = iota + k * 16
            for l in range(_CHUNK):
                vec = rows_v[b, l, pl.ds(k * 16, 16)]
                plsc.store_scatter(oblk, [dvec, jnp.full((16,), l,
                                                         jnp.int32)], vec)

    def store(c, b):
        pltpu.async_copy(
            oblk_v.at[b], out_hbm.at[:, pl.ds((base + c) * _CHUNK, _CHUNK)],
            ssem.at[b])

    def store_wait(b):
        pltpu.make_async_copy(
            out_hbm.at[:, pl.ds(0, _CHUNK)], oblk_v.at[b], ssem.at[b]).wait()

    prep_and_gather(0, 0)

    def body(c, carry):
        b = lax.rem(c, 2)
        nb = 1 - b

        @pl.when(c + 1 < _CPW)
        def _():
            prep_and_gather(c + 1, nb)

        gather_wait(b)

        @pl.when(c >= 2)
        def _():
            store_wait(b)

        transpose(c, b)
        store(c, b)
        return carry

    lax.fori_loop(0, _CPW, body, 0)
    store_wait(0)
    store_wait(1)


def _impl(node_feature, weight):
    idx = jnp.pad(node_feature[:, 0], (0, _N_PAD - _N))
    p = _repack(weight.T)
    return _emb_lookup(idx, p).T


def kernel(node_feature, weight):
    return jax.jit(_impl)(node_feature, weight)


# speedup vs baseline: 1.5225x; 1.3668x over previous
import functools

import jax
import jax.numpy as jnp
from jax import lax
from jax.experimental import pallas as pl
from jax.experimental.pallas import tpu as pltpu
from jax.experimental.pallas import tpu_sc as plsc

_N = 100000
_D = 64
_NC, _NS = 2, 16
_NW = _NC * _NS
_CHUNK = 128
_NCHUNK = 25
_NBUF = 12
_LEAD = 8
_W_ROWS = _CHUNK * _NCHUNK


@functools.partial(
    pl.kernel,
    out_type=jax.ShapeDtypeStruct((_N, _D), jnp.float32),
    mesh=plsc.VectorSubcoreMesh(core_axis_name="c", subcore_axis_name="s"),
    scratch_types=[
        pltpu.VMEM((_W_ROWS,), jnp.int32),
        pltpu.VMEM((_NBUF, _CHUNK, _D), jnp.float32),
        pltpu.SemaphoreType.DMA((_NBUF,)),
        pltpu.SemaphoreType.DMA((_NBUF,)),
    ],
    compiler_params=pltpu.CompilerParams(use_tc_tiling_on_sc=False),
)
def _emb_lookup(idx_hbm, table_hbm, out_hbm, idx_v, rows_v, gsem, ssem):
    wid = lax.axis_index("s") * _NC + lax.axis_index("c")
    base = jnp.minimum(wid * _W_ROWS, _N - _W_ROWS)
    pltpu.sync_copy(idx_hbm.at[pl.ds(base, _W_ROWS)], idx_v)

    def gather(j, b):
        pltpu.async_copy(
            table_hbm.at[idx_v.at[pl.ds(j * _CHUNK, _CHUNK)]],
            rows_v.at[b], gsem.at[b])

    def gather_wait(b):
        pltpu.make_async_copy(
            table_hbm.at[idx_v.at[pl.ds(0, _CHUNK)]],
            rows_v.at[b], gsem.at[b]).wait()

    def store(j, b):
        pltpu.async_copy(
            rows_v.at[b], out_hbm.at[pl.ds(base + j * _CHUNK, _CHUNK)],
            ssem.at[b])

    def store_wait(b):
        pltpu.make_async_copy(
            out_hbm.at[pl.ds(base, _CHUNK)], rows_v.at[b], ssem.at[b]).wait()

    for q in range(_LEAD):
        gather(q, q % _NBUF)
    for m in range(_NCHUNK):
        q = m + _LEAD
        if q < _NCHUNK:
            bq = q % _NBUF
            if q >= _NBUF:
                store_wait(bq)
            gather(q, bq)
        b = m % _NBUF
        gather_wait(b)
        store(m, b)
    for m in range(max(0, _NCHUNK - _NBUF), _NCHUNK):
        store_wait(m % _NBUF)


@jax.jit
def kernel(node_feature, weight):
    return _emb_lookup(node_feature[:, 0], weight)
